# baseline (speedup 1.0000x reference)
import functools

import jax
import jax.numpy as jnp
from jax import lax
from jax.experimental import pallas as pl
from jax.experimental.pallas import tpu as pltpu

B = 4; T = 2048; DM = 512; H = 8; DH = 64
NLOCAL = 4; NLSH = 4; BUCKET = 64; NHASH = 4; WINDOW = 128; L = 4
V_MCC = 400; V_AMT = 100; E_MCC = 256; E_AMT = 256; FF = 2048; HH = 512
N = B * T
NB = T // BUCKET
NCH = NHASH * NB
NBKT = NHASH * NB
NT = NHASH * T
BH = B * NLSH
NW = T // WINDOW
TB = 512
NTB = N // TB


def _embed_body(mcc_ref, amt_ref, em_ref, ea_ref, pw_ref, pb_ref, pe_ref, x_ref):
    mccb = mcc_ref[...]
    amtb = amt_ref[...]
    oh_m = (lax.broadcasted_iota(jnp.int32, (TB, V_MCC), 1) == mccb
            ).astype(jnp.float32)
    oh_a = (lax.broadcasted_iota(jnp.int32, (TB, V_AMT), 1) == amtb
            ).astype(jnp.float32)
    e_m = jnp.dot(oh_m, em_ref[...], preferred_element_type=jnp.float32)
    e_a = jnp.dot(oh_a, ea_ref[...], preferred_element_type=jnp.float32)
    e = jnp.concatenate([e_m, e_a], axis=1)
    x = jnp.dot(e, pw_ref[...], preferred_element_type=jnp.float32)
    x_ref[...] = x + pb_ref[...] + pe_ref[...]


def _embed(mcc, amt, emb_mcc, emb_amt, proj_W, proj_b, pe):
    return pl.pallas_call(
        _embed_body,
        grid=(NTB,),
        in_specs=[
            pl.BlockSpec((TB, 1), lambda i: (i, 0)),
            pl.BlockSpec((TB, 1), lambda i: (i, 0)),
            pl.BlockSpec((V_MCC, E_MCC), lambda i: (0, 0)),
            pl.BlockSpec((V_AMT, E_AMT), lambda i: (0, 0)),
            pl.BlockSpec((DM, DM), lambda i: (0, 0)),
            pl.BlockSpec((1, DM), lambda i: (0, 0)),
            pl.BlockSpec((TB, DM), lambda i: (i, 0)),
        ],
        out_specs=pl.BlockSpec((TB, DM), lambda i: (i, 0)),
        out_shape=jax.ShapeDtypeStruct((N, DM), jnp.float32),
    )(mcc, amt, emb_mcc, emb_amt, proj_W, proj_b, pe)


def _qkv_body(x_ref, qw_ref, vw_ref, qkh_ref, vh_ref):
    x = x_ref[...]
    qk = jnp.dot(x, qw_ref[...], preferred_element_type=jnp.float32)
    vv = jnp.dot(x, vw_ref[...], preferred_element_type=jnp.float32)
    for h in range(H):
        qkh_ref[0, h] = qk[:, h * DH:(h + 1) * DH]
        vh_ref[0, h] = vv[:, h * DH:(h + 1) * DH]


def _qkv(x, qw, vw):
    return pl.pallas_call(
        _qkv_body,
        grid=(NTB,),
        in_specs=[
            pl.BlockSpec((TB, DM), lambda i: (i, 0)),
            pl.BlockSpec((DM, DM), lambda i: (0, 0)),
            pl.BlockSpec((DM, DM), lambda i: (0, 0)),
        ],
        out_specs=[
            pl.BlockSpec((1, H, TB, DH), lambda i: (i // 4, 0, i % 4, 0)),
            pl.BlockSpec((1, H, TB, DH), lambda i: (i // 4, 0, i % 4, 0)),
        ],
        out_shape=[
            jax.ShapeDtypeStruct((B, H, T, DH), jnp.float32),
            jax.ShapeDtypeStruct((B, H, T, DH), jnp.float32),
        ],
    )(x, qw, vw)


def _local_body(q_ref, kc_ref, kp_ref, vc_ref, vp_ref, mc_ref, mp_ref, o_ref):
    w = pl.program_id(2)
    q = q_ref[0, 0]
    kc = kc_ref[0, 0]
    kp = kp_ref[0, 0]
    vc = vc_ref[0, 0]
    vp = jnp.where(w > 0, vp_ref[0, 0], 0.0)
    dn = (((1,), (1,)), ((), ()))
    dc = lax.dot_general(q, kc, dn, preferred_element_type=jnp.float32) * 0.125
    dp = lax.dot_general(q, kp, dn, preferred_element_type=jnp.float32) * 0.125
    ii = lax.broadcasted_iota(jnp.int32, (WINDOW, WINDOW), 0)
    jj = lax.broadcasted_iota(jnp.int32, (WINDOW, WINDOW), 1)
    mcur = (mc_ref[0] == 0) & (ii >= jj)
    mprev = (mp_ref[0] == 0) & (w > 0)
    dc = jnp.where(mcur, dc, -1e9)
    dp = jnp.where(mprev, dp, -1e9)
    d = jnp.concatenate([dp, dc], axis=1)
    m = jnp.max(d, axis=1, keepdims=True)
    e = jnp.exp(d - m)
    s = jnp.sum(e, axis=1, keepdims=True)
    v2 = jnp.concatenate([vp, vc], axis=0)
    o_ref[0, 0] = jnp.dot(e, v2, preferred_element_type=jnp.float32) / s


def _local_attn(qkh, vh, mcc3):
    cur = lambda b, h, w: (b, h, w, 0)
    prv = lambda b, h, w: (b, h, jnp.maximum(w - 1, 0), 0)
    return pl.pallas_call(
        _local_body,
        grid=(B, NLOCAL, NW),
        in_specs=[
            pl.BlockSpec((1, 1, WINDOW, DH), cur),
            pl.BlockSpec((1, 1, WINDOW, DH), cur),
            pl.BlockSpec((1, 1, WINDOW, DH), prv),
            pl.BlockSpec((1, 1, WINDOW, DH), cur),
            pl.BlockSpec((1, 1, WINDOW, DH), prv),
            pl.BlockSpec((1, 1, WINDOW), lambda b, h, w: (b * NW + w, 0, 0)),
            pl.BlockSpec((1, 1, WINDOW),
                         lambda b, h, w: (b * NW + jnp.maximum(w - 1, 0), 0, 0)),
        ],
        out_specs=pl.BlockSpec((1, 1, WINDOW, DH), cur),
        out_shape=jax.ShapeDtypeStruct((B, NLOCAL, T, DH), jnp.float32),
    )(qkh, qkh, qkh, vh, vh, mcc3, mcc3)


def _bucket_body(qk_ref, rot_ref, b_ref):
    rx = jnp.dot(qk_ref[0, 0], rot_ref[0], preferred_element_type=jnp.float32)
    sc = jnp.concatenate([rx, -rx], axis=1)
    m = jnp.max(sc, axis=1, keepdims=True)
    io = lax.broadcasted_iota(jnp.int32, sc.shape, 1)
    cand = jnp.where(sc == m, io, NB)
    b_ref[0] = jnp.min(cand, axis=1, keepdims=True)


CS = 512
NCS = NT // CS


def _rank_body(b_ref, rank_ref, w_scr):
    iob = lax.broadcasted_iota(jnp.int32, (NBKT, CS), 0)
    ior = lax.broadcasted_iota(jnp.int32, (CS, CS), 0)
    ioc = lax.broadcasted_iota(jnp.int32, (CS, CS), 1)
    U = (ior < ioc).astype(jnp.float32)

    def pass1(c, prevc):
        vals = b_ref[0, :, pl.ds(c * CS, CS)]
        O = (iob == vals).astype(jnp.float32)
        C = jnp.dot(O, U, preferred_element_type=jnp.float32)
        within = jnp.sum(C * O, axis=0, keepdims=True)
        prev_ex = jnp.sum(prevc * O, axis=0, keepdims=True)
        w_scr[pl.ds(c, 1), :] = within + prev_ex
        return prevc + jnp.sum(O, axis=1, keepdims=True)

    totals = lax.fori_loop(0, NCS, pass1, jnp.zeros((NBKT, 1), jnp.float32))
    iobr = lax.broadcasted_iota(jnp.int32, (NBKT, NBKT), 0)
    iobc = lax.broadcasted_iota(jnp.int32, (NBKT, NBKT), 1)
    Ls = (iobc < iobr).astype(jnp.float32)
    offsets = jnp.dot(Ls, totals, preferred_element_type=jnp.float32)

    def pass2(c, _):
        vals = b_ref[0, :, pl.ds(c * CS, CS)]
        O = (iob == vals).astype(jnp.float32)
        off = jnp.sum(offsets * O, axis=0, keepdims=True)
        rank_ref[0, :, pl.ds(c * CS, CS)] = (
            off + w_scr[pl.ds(c, 1), :]).astype(jnp.int32)
        return 0

    lax.fori_loop(0, NCS, pass2, 0)


def _ranks(buckets):
    return pl.pallas_call(
        _rank_body,
        grid=(BH,),
        in_specs=[pl.BlockSpec((1, 1, NT), lambda i: (i, 0, 0))],
        out_specs=pl.BlockSpec((1, 1, NT), lambda i: (i, 0, 0)),
        out_shape=jax.ShapeDtypeStruct((BH, 1, NT), jnp.int32),
        scratch_shapes=[pltpu.VMEM((NCS, CS), jnp.float32)],
    )(buckets)


def _lsh_body(qc_ref, qp_ref, vc_ref, vp_ref, stc_ref, stp_ref, stq_ref,
              smc_ref, smp_ref, so_ref, sl_ref):
    q = qc_ref[0, 0]
    k2 = jnp.concatenate([qc_ref[0, 0], qp_ref[0, 0]], axis=0)
    nrm = jnp.sqrt(jnp.sum(k2 * k2, axis=1, keepdims=True))
    kn = k2 / jnp.maximum(nrm, 1e-9)
    dn = (((1,), (1,)), ((), ()))
    d = lax.dot_general(q, kn, dn, preferred_element_type=jnp.float32) * 0.125
    smkv = jnp.concatenate([smc_ref[0, 0], smp_ref[0, 0]], axis=1)
    bkv = jnp.concatenate([stc_ref[0, 0], stp_ref[0, 0]], axis=1)
    bq = stq_ref[0, 0]
    d = jnp.where(smkv > 0, d, -1e9)
    d = jnp.where(bq < bkv, -1e9, d)
    d = jnp.where(bq == bkv, -5e4, d)
    m = jnp.max(d, axis=1, keepdims=True)
    e = jnp.exp(d - m)
    s = jnp.sum(e, axis=1, keepdims=True)
    lse = jnp.log(s) + m
    sl_ref[0, 0] = lse
    p = jnp.exp(d - lse)
    v2 = jnp.concatenate([vc_ref[0, 0], vp_ref[0, 0]], axis=0)
    so_ref[0, 0] = jnp.dot(p, v2, preferred_element_type=jnp.float32)


def _lsh_attn(sqk, sv, st_row, st_col, sm_row):
    prev = lambda bh, c: (bh, (c + NCH - 1) % NCH, 0, 0)
    cur = lambda bh, c: (bh, c, 0, 0)
    return pl.pallas_call(
        _lsh_body,
        grid=(BH, NCH),
        in_specs=[
            pl.BlockSpec((1, 1, BUCKET, DH), cur),
            pl.BlockSpec((1, 1, BUCKET, DH), prev),
            pl.BlockSpec((1, 1, BUCKET, DH), cur),
            pl.BlockSpec((1, 1, BUCKET, DH), prev),
            pl.BlockSpec((1, 1, 1, BUCKET), cur),
            pl.BlockSpec((1, 1, 1, BUCKET), prev),
            pl.BlockSpec((1, 1, BUCKET, 1), cur),
            pl.BlockSpec((1, 1, 1, BUCKET), cur),
            pl.BlockSpec((1, 1, 1, BUCKET), prev),
        ],
        out_specs=[
            pl.BlockSpec((1, 1, BUCKET, DH), cur),
            pl.BlockSpec((1, 1, BUCKET, 1), cur),
        ],
        out_shape=[
            jax.ShapeDtypeStruct((BH, NCH, BUCKET, DH), jnp.float32),
            jax.ShapeDtypeStruct((BH, NCH, BUCKET, 1), jnp.float32),
        ],
    )(sqk, sqk, sv, sv, st_row, st_row, st_col, sm_row, sm_row)


def _combine_body(ou_ref, lg_ref, o_ref):
    lgs = [lg_ref[0, h] for h in range(NHASH)]
    m = lgs[0]
    for h in range(1, NHASH):
        m = jnp.maximum(m, lgs[h])
    es = [jnp.exp(g - m) for g in lgs]
    s = es[0]
    for h in range(1, NHASH):
        s = s + es[h]
    lse = jnp.log(s) + m
    o = ou_ref[0, 0] * jnp.exp(lgs[0] - lse)
    for h in range(1, NHASH):
        o = o + ou_ref[0, h] * jnp.exp(lgs[h] - lse)
    o_ref[0, 0] = o


def _lsh_combine(ou, lg):
    return pl.pallas_call(
        _combine_body,
        grid=(BH, T // TB),
        in_specs=[
            pl.BlockSpec((1, NHASH, TB, DH), lambda bh, t: (bh, 0, t, 0)),
            pl.BlockSpec((1, NHASH, TB, 1), lambda bh, t: (bh, 0, t, 0)),
        ],
        out_specs=pl.BlockSpec((1, 1, TB, DH),
                               lambda bh, t: (bh // NLSH, bh % NLSH, t, 0)),
        out_shape=jax.ShapeDtypeStruct((B, NLSH, T, DH), jnp.float32),
    )(ou, lg)


def _ln(h, g, b):
    mu = jnp.mean(h, axis=1, keepdims=True)
    xc = h - mu
    var = jnp.mean(xc * xc, axis=1, keepdims=True)
    return xc * lax.rsqrt(var + 1e-5) * g + b


def _attnout_body(l0, l1, l2, l3, s0, s1, s2, s3, x_ref, w_ref, b_ref,
                  g_ref, bb_ref, o_ref):
    attn = jnp.concatenate(
        [r[0, 0] for r in (l0, l1, l2, l3, s0, s1, s2, s3)], axis=1)
    h = jnp.dot(attn, w_ref[...], preferred_element_type=jnp.float32)
    h = h + b_ref[...] + x_ref[...]
    o_ref[...] = _ln(h, g_ref[...], bb_ref[...])


def _attnout(lo, lsho, x, w, b, g, bb):
    def hspec(h):
        return pl.BlockSpec((1, 1, TB, DH), lambda i, h=h: (i // 4, h, i % 4, 0))
    return pl.pallas_call(
        _attnout_body,
        grid=(NTB,),
        in_specs=[hspec(0), hspec(1), hspec(2), hspec(3),
                  hspec(0), hspec(1), hspec(2), hspec(3),
                  pl.BlockSpec((TB, DM), lambda i: (i, 0)),
                  pl.BlockSpec((DM, DM), lambda i: (0, 0)),
                  pl.BlockSpec((1, DM), lambda i: (0, 0)),
                  pl.BlockSpec((1, DM), lambda i: (0, 0)),
                  pl.BlockSpec((1, DM), lambda i: (0, 0))],
        out_specs=pl.BlockSpec((TB, DM), lambda i: (i, 0)),
        out_shape=jax.ShapeDtypeStruct((N, DM), jnp.float32),
    )(lo, lo, lo, lo, lsho, lsho, lsho, lsho, x, w, b, g, bb)


def _gelu(x):
    return 0.5 * x * (1.0 + lax.erf(x * 0.7071067811865476))


def _ffn_body(x_ref, w1_ref, b1_ref, w2_ref, b2_ref, g_ref, bb_ref, o_ref):
    x = x_ref[...]
    h = jnp.dot(x, w1_ref[...], preferred_element_type=jnp.float32) + b1_ref[...]
    h = _gelu(h)
    h = jnp.dot(h, w2_ref[...], preferred_element_type=jnp.float32) + b2_ref[...]
    o_ref[...] = _ln(h + x, g_ref[...], bb_ref[...])


def _ffn(x, w1, b1, w2, b2, g, bb):
    return pl.pallas_call(
        _ffn_body,
        grid=(NTB,),
        in_specs=[
            pl.BlockSpec((TB, DM), lambda i: (i, 0)),
            pl.BlockSpec((DM, FF), lambda i: (0, 0)),
            pl.BlockSpec((1, FF), lambda i: (0, 0)),
            pl.BlockSpec((FF, DM), lambda i: (0, 0)),
            pl.BlockSpec((1, DM), lambda i: (0, 0)),
            pl.BlockSpec((1, DM), lambda i: (0, 0)),
            pl.BlockSpec((1, DM), lambda i: (0, 0)),
        ],
        out_specs=pl.BlockSpec((TB, DM), lambda i: (i, 0)),
        out_shape=jax.ShapeDtypeStruct((N, DM), jnp.float32),
    )(x, w1, b1, w2, b2, g, bb)


def _heads_body(x_ref, mw1_ref, mb1_ref, mw2_ref, mb2_ref,
                aw1_ref, ab1_ref, aw2_ref, ab2_ref, lm_ref, la_ref):
    x = x_ref[...]
    hm = _gelu(jnp.dot(x, mw1_ref[...], preferred_element_type=jnp.float32)
               + mb1_ref[...])
    lm_ref[...] = jnp.dot(hm, mw2_ref[...],
                          preferred_element_type=jnp.float32) + mb2_ref[...]
    ha = _gelu(jnp.dot(x, aw1_ref[...], preferred_element_type=jnp.float32)
               + ab1_ref[...])
    la_ref[...] = jnp.dot(ha, aw2_ref[...],
                          preferred_element_type=jnp.float32) + ab2_ref[...]


def _heads(x, mw1, mb1, mw2, mb2, aw1, ab1, aw2, ab2):
    return pl.pallas_call(
        _heads_body,
        grid=(NTB,),
        in_specs=[
            pl.BlockSpec((TB, DM), lambda i: (i, 0)),
            pl.BlockSpec((DM, HH), lambda i: (0, 0)),
            pl.BlockSpec((1, HH), lambda i: (0, 0)),
            pl.BlockSpec((HH, V_MCC), lambda i: (0, 0)),
            pl.BlockSpec((1, V_MCC), lambda i: (0, 0)),
            pl.BlockSpec((DM, HH), lambda i: (0, 0)),
            pl.BlockSpec((1, HH), lambda i: (0, 0)),
            pl.BlockSpec((HH, V_AMT), lambda i: (0, 0)),
            pl.BlockSpec((1, V_AMT), lambda i: (0, 0)),
        ],
        out_specs=[
            pl.BlockSpec((TB, V_MCC), lambda i: (i, 0)),
            pl.BlockSpec((TB, V_AMT), lambda i: (i, 0)),
        ],
        out_shape=[
            jax.ShapeDtypeStruct((N, V_MCC), jnp.float32),
            jax.ShapeDtypeStruct((N, V_AMT), jnp.float32),
        ],
    )(x, mw1, mb1, mw2, mb2, aw1, ab1, aw2, ab2)


def kernel(mcc, amount_bin, emb_mcc, emb_amt, proj_W, proj_b, qk_W, v_W,
           out_W, out_b, ff1_W, ff1_b, ff2_W, ff2_b, ln1_g, ln1_b, ln2_g,
           ln2_b, hm_W1, hm_b1, hm_W2, hm_b2, ha_W1, ha_b1, ha_W2, ha_b2):
    mcc = mcc.astype(jnp.int32)
    amount_bin = amount_bin.astype(jnp.int32)

    pos = jnp.arange(T)[:, None].astype(jnp.float32)
    div = jnp.exp(jnp.arange(0, DM, 2).astype(jnp.float32) * (-9.21034037198 / DM))
    pe = jnp.zeros((T, DM))
    pe = pe.at[:, 0::2].set(jnp.sin(pos * div))
    pe = pe.at[:, 1::2].set(jnp.cos(pos * div))
    pe_full = jnp.broadcast_to(pe[None], (B, T, DM)).reshape(N, DM)
    rkey = jax.random.key(42)
    rots = jnp.stack([
        jax.random.normal(jax.random.fold_in(rkey, l), (DH, NHASH, NB // 2))
        for l in range(L)])
    rots = rots.transpose(0, 2, 1, 3)

    mcc3 = mcc.reshape(B * NW, 1, WINDOW)
    x = _embed(mcc.reshape(N, 1), amount_bin.reshape(N, 1),
               emb_mcc, emb_amt, proj_W, proj_b[None], pe_full)

    for l in range(L):
        qkh, vh = _qkv(x, qk_W[l], v_W[l])
        lo = _local_attn(qkh, vh, mcc3)

        bc = pl.pallas_call(
            _bucket_body,
            grid=(B, NLSH, NHASH),
            in_specs=[
                pl.BlockSpec((1, 1, T, DH), lambda b, h, j: (b, NLOCAL + h, 0, 0)),
                pl.BlockSpec((1, DH, NB // 2), lambda b, h, j: (j, 0, 0)),
            ],
            out_specs=pl.BlockSpec(
                (1, T, 1), lambda b, h, j: ((b * NLSH + h) * NHASH + j, 0, 0)),
            out_shape=jax.ShapeDtypeStruct((BH * NHASH, T, 1), jnp.int32),
        )(qkh, rots[l])
        bl = bc.reshape(BH, NHASH, T) + (jnp.arange(NHASH) * NB)[None, :, None]
        buckets = bl.reshape(BH, 1, NT)

        rank = _ranks(buckets).reshape(BH, NT)

        iota_r = jnp.arange(NT, dtype=jnp.int32)[None, :]
        sticker = jnp.zeros((BH, NT), jnp.int32).at[
            jnp.arange(BH)[:, None], rank].set(
                jnp.broadcast_to(iota_r, (BH, NT)))
        st = sticker % T
        qk_lsh = qkh[:, NLOCAL:].reshape(BH, T, DH)
        v_lsh = vh[:, NLOCAL:].reshape(BH, T, DH)
        sqk = jnp.take_along_axis(qk_lsh, st[..., None], axis=1)
        sv = jnp.take_along_axis(v_lsh, st[..., None], axis=1)
        mask_h = jnp.repeat(mcc == 0, NLSH, axis=0)
        sm = jnp.take_along_axis(mask_h, st, axis=1).astype(jnp.float32)

        sqk4 = sqk.reshape(BH, NCH, BUCKET, DH)
        sv4 = sv.reshape(BH, NCH, BUCKET, DH)
        st_row = st.reshape(BH, NCH, 1, BUCKET)
        st_col = st.reshape(BH, NCH, BUCKET, 1)
        sm_row = sm.reshape(BH, NCH, 1, BUCKET)
        so, sl = _lsh_attn(sqk4, sv4, st_row, st_col, sm_row)

        so2 = so.reshape(BH, NT, DH)
        sl2 = sl.reshape(BH, NT)
        ou = jnp.take_along_axis(so2, rank[..., None], axis=1)
        lg = jnp.take_along_axis(sl2, rank, axis=1)
        ou4 = ou.reshape(BH, NHASH, T, DH)
        lg4 = lg.reshape(BH, NHASH, T, 1)
        lsho = _lsh_combine(ou4, lg4)

        x = _attnout(lo, lsho, x, out_W[l], out_b[l][None], ln1_g[l][None],
                     ln1_b[l][None])
        x = _ffn(x, ff1_W[l], ff1_b[l][None], ff2_W[l], ff2_b[l][None],
                 ln2_g[l][None], ln2_b[l][None])

    lm, la = _heads(x, hm_W1, hm_b1[None], hm_W2, hm_b2[None],
                    ha_W1, ha_b1[None], ha_W2, ha_b2[None])
    return lm.reshape(B, T, V_MCC), la.reshape(B, T, V_AMT)
